# baseline (speedup 1.0000x reference)
import functools

import jax
import jax.numpy as jnp
from jax import lax
from jax.experimental import pallas as pl
from jax.experimental.pallas import tpu as pltpu
from jax.experimental.pallas import tpu_sc as plsc

N_ROIS = 20000
LOSS_DIM = 84
KEEP = 512
NW = 32
RPW = N_ROIS // NW
WIN = 632
PASS_ROWS = 320
GROUPS = 40
OUT_W = 640
F32_INF_BITS = 0x7F800000

_sc_mesh = plsc.VectorSubcoreMesh(core_axis_name="c", subcore_axis_name="s")


@functools.partial(
    pl.kernel,
    mesh=_sc_mesh,
    out_type=jax.ShapeDtypeStruct((NW, OUT_W * 16), jnp.float32),
    scratch_types=[
        pltpu.VMEM((PASS_ROWS, LOSS_DIM), jnp.float32),
        pltpu.VMEM((PASS_ROWS, LOSS_DIM), jnp.float32),
        pltpu.VMEM((OUT_W * 16,), jnp.float32),
    ],
)
def _sc_losses(t_hbm, p_hbm, out_hbm, tbuf, pbuf, lbuf):
    wid = lax.axis_index("s") * 2 + lax.axis_index("c")
    base = 8 * ((RPW * wid) // 8)
    nxt = 8 * ((RPW * (wid + 1)) // 8)
    r_mine = nxt - base

    lane = lax.iota(jnp.int32, 16)
    offs = (0, 16, 32, 48, 64, 68)
    zero = jnp.zeros((16,), jnp.float32)

    def smooth_l1(xt, xp):
        d = jnp.abs(xt - xp)
        m = jnp.minimum(d, 1.0)
        return m * (d - 0.5 * m)

    for p in range(2):
        rstart = p * 312
        pltpu.sync_copy(t_hbm.at[pl.ds(base + rstart, PASS_ROWS), :], tbuf)
        pltpu.sync_copy(p_hbm.at[pl.ds(base + rstart, PASS_ROWS), :], pbuf)

        def group(g, carry):
            for r16 in range(16):
                prow = g * 16 + r16
                grow = rstart + prow
                acc = zero
                for ci, off in enumerate(offs):
                    x = smooth_l1(tbuf[prow, pl.ds(off, 16)],
                                  pbuf[prow, pl.ds(off, 16)])
                    if ci == 5:
                        x = jnp.where(lane >= 12, x, 0.0)
                    acc = acc + x
                acc = jnp.where(grow < r_mine, acc, zero)
                lbuf[pl.ds(grow * 16, 16)] = acc
            return carry

        lax.fori_loop(0, PASS_ROWS // 16, group, 0)

    for i in range(632, OUT_W):
        lbuf[pl.ds(i * 16, 16)] = zero

    pltpu.sync_copy(lbuf, out_hbm.at[wid])


def _select_body(v_ref, out_ref):
    vals = v_ref[...]
    bits = lax.bitcast_convert_type(vals, jnp.int32)

    def count_ge(m):
        return jnp.sum(jnp.where(bits >= m, 1, 0))

    def body(_, carry):
        lo, hi = carry
        q = jnp.maximum((hi - lo) // 4, 1)
        m1 = lo + q
        m2 = lo + 2 * q
        m3 = lo + 3 * q
        c1 = count_ge(m1) >= KEEP
        c2 = count_ge(m2) >= KEEP
        c3 = count_ge(m3) >= KEEP
        lo2 = jnp.where(c3, m3, jnp.where(c2, m2, jnp.where(c1, m1, lo)))
        hi2 = jnp.where(c1, jnp.where(c2, jnp.where(c3, hi, m3), m2), m1)
        return lo2, hi2

    lo, hi = lax.fori_loop(
        0, 16, body, (jnp.int32(0), jnp.int32(F32_INF_BITS)))
    t_val = lax.bitcast_convert_type(lo, jnp.float32)
    gt = bits > lo
    cnt_gt = jnp.sum(jnp.where(gt, 1, 0))
    sum_gt = jnp.sum(jnp.where(gt, vals, 0.0))
    res = (sum_gt + (KEEP - cnt_gt).astype(jnp.float32) * t_val) / KEEP
    out_ref[0, 0] = res


@jax.jit
def _ohem(target, predict):
    part = _sc_losses(target, predict)
    losses = jnp.sum(part.reshape(NW, OUT_W, 16), axis=2)
    out = pl.pallas_call(
        _select_body,
        out_specs=pl.BlockSpec(memory_space=pltpu.SMEM),
        out_shape=jax.ShapeDtypeStruct((1, 1), jnp.float32),
    )(losses)
    return out[0, 0]


def kernel(target, predict):
    return _ohem(target, predict)
